# baseline (speedup 1.0000x reference)
import jax
import jax.numpy as jnp
from jax import lax
from jax.experimental import pallas as pl
from jax.experimental.pallas import tpu as pltpu
from jax.experimental.pallas import tpu_sc as plsc

CHAR_VOCAB = 1000
WORD_VOCAB = 1000000
OUT_DIM = 64
PAD = 128
B, L = 4096, 200
N = B * L

_NC, _NS = 2, 16
NW = _NC * _NS
_NBUF = 2
_LSPLIT = 120


def _make_sc_body(rows_per_w):
    per_w = rows_per_w * L
    n_groups = rows_per_w // _NBUF

    def _sc_body(word_hbm, char_hbm, widx_hbm, cidx_hbm, wem_hbm, cem_hbm,
                 widx_v, cidx_v, wrows_v, crows_v, char_sp,
                 semw0, semw1, semc0, semc1,
                 semow0, semow1, semoc0, semoc1):
        semw = [semw0, semw1]
        semc = [semc0, semc1]
        semow = [semow0, semow1]
        semoc = [semoc0, semoc1]
        wid = lax.axis_index("s") * _NC + lax.axis_index("c")
        t0 = wid * per_w

        @pl.when(lax.axis_index("s") == 0)
        def _():
            pltpu.sync_copy(char_hbm, char_sp)
        plsc.subcore_barrier()

        pltpu.sync_copy(widx_hbm.at[pl.ds(t0, per_w)], widx_v)
        pltpu.sync_copy(cidx_hbm.at[pl.ds(t0, per_w)], cidx_v)

        lsl = [(0, _LSPLIT), (_LSPLIT, L - _LSPLIT)]

        def group(g, carry):
            i0 = g * _NBUF

            @pl.when(g > 0)
            def _():
                for p in range(_NBUF):
                    pltpu.make_async_copy(
                        wrows_v.at[p], wem_hbm.at[pl.ds(t0, L)],
                        semow[p]).wait()
                    pltpu.make_async_copy(
                        crows_v.at[p], cem_hbm.at[pl.ds(t0, L)],
                        semoc[p]).wait()

            gw = []
            gc = []
            for p in range(_NBUF):
                for (o, n) in lsl:
                    gw.append(pltpu.async_copy(
                        word_hbm.at[widx_v.at[pl.ds((i0 + p) * L + o, n)]],
                        wrows_v.at[p, pl.ds(o, n)], semw[p]))
                    gc.append(pltpu.async_copy(
                        char_sp.at[cidx_v.at[pl.ds((i0 + p) * L + o, n)]],
                        crows_v.at[p, pl.ds(o, n)], semc[p]))
            for p in range(_NBUF):
                gw[2 * p].wait()
                gw[2 * p + 1].wait()
                pltpu.async_copy(
                    wrows_v.at[p],
                    wem_hbm.at[pl.ds(t0 + (i0 + p) * L, L)], semow[p])
            for p in range(_NBUF):
                gc[2 * p].wait()
                gc[2 * p + 1].wait()
                pltpu.async_copy(
                    crows_v.at[p],
                    cem_hbm.at[pl.ds(t0 + (i0 + p) * L, L)], semoc[p])
            return carry

        lax.fori_loop(0, n_groups, group, 0)

        for p in range(_NBUF):
            pltpu.make_async_copy(
                wrows_v.at[p], wem_hbm.at[pl.ds(t0, L)], semow[p]).wait()
            pltpu.make_async_copy(
                crows_v.at[p], cem_hbm.at[pl.ds(t0, L)], semoc[p]).wait()

    return _sc_body, per_w


def _sc_gather(word_table, char_table, widx, cidx, n_rows):
    rows_per_w = n_rows // NW
    body, per_w = _make_sc_body(rows_per_w)
    n_tok = n_rows * L
    mesh = plsc.VectorSubcoreMesh(core_axis_name="c", subcore_axis_name="s")
    return pl.kernel(
        body,
        out_type=(jax.ShapeDtypeStruct((n_tok, OUT_DIM), jnp.float32),
                  jax.ShapeDtypeStruct((n_tok, OUT_DIM), jnp.float32)),
        mesh=mesh,
        scratch_types=[
            pltpu.VMEM((per_w,), jnp.int32),
            pltpu.VMEM((per_w,), jnp.int32),
            pltpu.VMEM((_NBUF, L, OUT_DIM), jnp.float32),
            pltpu.VMEM((_NBUF, L, OUT_DIM), jnp.float32),
            pltpu.VMEM_SHARED((CHAR_VOCAB, OUT_DIM), jnp.float32),
        ] + [pltpu.SemaphoreType.DMA] * (4 * _NBUF),
        compiler_params=pltpu.CompilerParams(use_tc_tiling_on_sc=False),
    )(word_table, char_table, widx, cidx)


_MIX_BLOCK = 3200


def _mix_body(w2_ref, c2_ref, w_ref, out_ref):
    w = w_ref[...]
    z = jnp.zeros((OUT_DIM, OUT_DIM), jnp.float32)
    wbig = jnp.concatenate([
        jnp.concatenate([w, z], axis=1),
        jnp.concatenate([z, w], axis=1),
    ], axis=0)
    out_ref[...] = (
        jnp.dot(w2_ref[...], wbig, preferred_element_type=jnp.float32)
        + c2_ref[...])


def _tc_mix(wem2, cem2, W_dense, n_pairs):
    return pl.pallas_call(
        _mix_body,
        grid=(n_pairs // _MIX_BLOCK,),
        in_specs=[
            pl.BlockSpec((_MIX_BLOCK, PAD), lambda i: (i, 0)),
            pl.BlockSpec((_MIX_BLOCK, PAD), lambda i: (i, 0)),
            pl.BlockSpec((OUT_DIM, OUT_DIM), lambda i: (0, 0)),
        ],
        out_specs=pl.BlockSpec((_MIX_BLOCK, PAD), lambda i: (i, 0)),
        out_shape=jax.ShapeDtypeStruct((n_pairs, PAD), jnp.float32),
    )(wem2, cem2, W_dense)


def kernel(char_id, word_id, char_table, word_table, W_dense):
    widx = word_id.reshape(N).astype(jnp.int32)
    cidx = char_id.reshape(N).astype(jnp.int32)
    half = B // 2
    ht = half * L
    outs = []
    for h in range(2):
        wem, cem = _sc_gather(word_table, char_table,
                              widx[h * ht:(h + 1) * ht],
                              cidx[h * ht:(h + 1) * ht], half)
        mix2 = _tc_mix(wem.reshape(ht // 2, PAD), cem.reshape(ht // 2, PAD),
                       W_dense, ht // 2)
        outs.append(mix2.reshape(half, L, OUT_DIM))
    return jnp.concatenate(outs, axis=0)
